# baseline (speedup 1.0000x reference)
import functools

import jax
import jax.numpy as jnp
from jax import lax
from jax.experimental import pallas as pl
from jax.experimental.pallas import tpu as pltpu
from jax.experimental.pallas import tpu_sc as plsc

B = 16384
L = 200
D = 32
NC = 2
NS = 16
NW = NC * NS
ROWS_PER_W = B // NW
C = 8
CHUNKS = ROWS_PER_W // C
SUB = 2
LSUB = L // SUB
UN = 8


def _sc_lookup_mean(x_r, table):
    mesh = plsc.VectorSubcoreMesh(core_axis_name="c", subcore_axis_name="s")

    @functools.partial(
        pl.kernel,
        out_type=jax.ShapeDtypeStruct((B, D), jnp.float32),
        mesh=mesh,
        scratch_types=[
            pltpu.VMEM((SUB * C, LSUB), jnp.int32),
            pltpu.VMEM((SUB * C, LSUB), jnp.int32),
            pltpu.VMEM((C * L, D), jnp.float32),
            pltpu.VMEM((C * L, D), jnp.float32),
            pltpu.VMEM((C, D), jnp.float32),
            pltpu.SemaphoreType.DMA,
            pltpu.SemaphoreType.DMA,
        ],
        compiler_params=pltpu.CompilerParams(use_tc_tiling_on_sc=False),
    )
    def body(x_hbm, tbl_hbm, out_hbm, idx0, idx1, gbuf0, gbuf1, obuf,
             sem0, sem1):
        wid = lax.axis_index("s") * NC + lax.axis_index("c")
        base = wid * ROWS_PER_W
        scale = jnp.float32(1.0 / L)

        def fire(gq, idx_v, gbuf, sem):
            row0 = base + gq * C
            pltpu.sync_copy(x_hbm.at[pl.ds(SUB * row0, SUB * C), :], idx_v)
            for j in range(SUB * C):
                pltpu.async_copy(
                    tbl_hbm.at[pl.ds(j * LSUB, LSUB), :],
                    gbuf.at[pl.ds(j * LSUB, LSUB), :],
                    sem)

        def drain(idx_v, gbuf, sem):
            for j in range(SUB * C):
                pltpu.make_async_copy(
                    tbl_hbm.at[pl.ds(j * LSUB, LSUB), :],
                    gbuf.at[pl.ds(j * LSUB, LSUB), :],
                    sem).wait()

        def reduce_store(gq, gbuf):
            row0 = base + gq * C
            for c in range(C):
                def step(jj, acc, c=c):
                    a0, a1, b0, b1 = acc
                    r = c * L + jj * UN
                    for u in range(0, UN, 2):
                        a0 = a0 + gbuf[r + u, pl.ds(0, 16)]
                        a1 = a1 + gbuf[r + u, pl.ds(16, 16)]
                        b0 = b0 + gbuf[r + u + 1, pl.ds(0, 16)]
                        b1 = b1 + gbuf[r + u + 1, pl.ds(16, 16)]
                    return (a0, a1, b0, b1)
                z = jnp.zeros((16,), jnp.float32)
                a0, a1, b0, b1 = (z, z, z, z)
                obuf[c, pl.ds(0, 16)] = (a0 + b0) * scale
                obuf[c, pl.ds(16, 16)] = (a1 + b1) * scale
            pltpu.sync_copy(obuf, out_hbm.at[pl.ds(row0, C), :])

        fire(0, idx0, gbuf0, sem0)

        def pair(k, carry):
            c0 = 2 * k
            c1 = 2 * k + 1
            fire(c1, idx1, gbuf1, sem1)
            drain(idx0, gbuf0, sem0)
            reduce_store(c0, gbuf0)

            @pl.when(c1 + 1 < CHUNKS)
            def _():
                fire(c1 + 1, idx0, gbuf0, sem0)

            drain(idx1, gbuf1, sem1)
            reduce_store(c1, gbuf1)
            return carry

        lax.fori_loop(0, CHUNKS // 2, pair, 0)

    return body(x_r, table)


def kernel(x, table):
    x_r = x.reshape(B * SUB, LSUB)
    return _sc_lookup_mean(x_r, table)


# speedup vs baseline: 1.3993x; 1.3993x over previous
import functools

import jax
import jax.numpy as jnp
from jax import lax
from jax.experimental import pallas as pl
from jax.experimental.pallas import tpu as pltpu
from jax.experimental.pallas import tpu_sc as plsc

B = 16384
L = 200
D = 32
NC = 2
NS = 16
NW = NC * NS
ROWS_PER_W = B // NW
C = 8
CHUNKS = ROWS_PER_W // C
CL = C * L
UN = 8


def _sc_lookup_mean(x_flat, table):
    mesh = plsc.VectorSubcoreMesh(core_axis_name="c", subcore_axis_name="s")

    @functools.partial(
        pl.kernel,
        out_type=jax.ShapeDtypeStruct((B, D), jnp.float32),
        mesh=mesh,
        scratch_types=[
            pltpu.VMEM((CL,), jnp.int32),
            pltpu.VMEM((CL,), jnp.int32),
            pltpu.VMEM((CL, D), jnp.float32),
            pltpu.VMEM((CL, D), jnp.float32),
            pltpu.VMEM((C, D), jnp.float32),
            pltpu.SemaphoreType.DMA,
            pltpu.SemaphoreType.DMA,
        ],
        compiler_params=pltpu.CompilerParams(use_tc_tiling_on_sc=False),
    )
    def body(x_hbm, tbl_hbm, out_hbm, idx0, idx1, gbuf0, gbuf1, obuf,
             sem0, sem1):
        wid = lax.axis_index("s") * NC + lax.axis_index("c")
        base = wid * ROWS_PER_W
        scale = jnp.float32(1.0 / L)

        def fire(gq, idx_v, gbuf, sem):
            row0 = base + gq * C
            pltpu.sync_copy(x_hbm.at[pl.ds(row0 * L, CL)], idx_v)
            pltpu.async_copy(tbl_hbm.at[idx_v], gbuf, sem)

        def drain(idx_v, gbuf, sem):
            pltpu.make_async_copy(tbl_hbm.at[idx_v], gbuf, sem).wait()

        def reduce_store(gq, gbuf):
            row0 = base + gq * C
            for c in range(C):
                def step(jj, acc, c=c):
                    a0, a1, b0, b1 = acc
                    r = c * L + jj * UN
                    for u in range(0, UN, 2):
                        a0 = a0 + gbuf[r + u, pl.ds(0, 16)]
                        a1 = a1 + gbuf[r + u, pl.ds(16, 16)]
                        b0 = b0 + gbuf[r + u + 1, pl.ds(0, 16)]
                        b1 = b1 + gbuf[r + u + 1, pl.ds(16, 16)]
                    return (a0, a1, b0, b1)
                z = jnp.zeros((16,), jnp.float32)
                a0, a1, b0, b1 = lax.fori_loop(0, L // UN, step, (z, z, z, z))
                obuf[c, pl.ds(0, 16)] = (a0 + b0) * scale
                obuf[c, pl.ds(16, 16)] = (a1 + b1) * scale
            pltpu.sync_copy(obuf, out_hbm.at[pl.ds(row0, C), :])

        fire(0, idx0, gbuf0, sem0)

        def pair(k, carry):
            c0 = 2 * k
            c1 = 2 * k + 1
            fire(c1, idx1, gbuf1, sem1)
            drain(idx0, gbuf0, sem0)
            reduce_store(c0, gbuf0)

            @pl.when(c1 + 1 < CHUNKS)
            def _():
                fire(c1 + 1, idx0, gbuf0, sem0)

            drain(idx1, gbuf1, sem1)
            reduce_store(c1, gbuf1)
            return carry

        lax.fori_loop(0, CHUNKS // 2, pair, 0)

    return body(x_flat, table)


def kernel(x, table):
    x_flat = x.reshape(B * L)
    return _sc_lookup_mean(x_flat, table)
